# baseline (speedup 1.0000x reference)
import functools

import jax
import jax.numpy as jnp
from jax import lax
from jax.experimental import pallas as pl
from jax.experimental.pallas import tpu as pltpu
from jax.experimental.pallas import tpu_sc as plsc

_N_NODES = 50_000
_E = 800_000
_NLG = 800_000
_LANES = 16
_NW = 32
_BLK = 256
_PAIRS = _BLK // 2
_NBLK_G = _E // _BLK
_NBLK_H = _NLG // _BLK
_ITERS_G = (_NBLK_G + _NW - 1) // _NW
_ITERS_H = (_NBLK_H + _NW - 1) // _NW
_PQ_WORDS = _E // _LANES


def _vmesh():
    return plsc.VectorSubcoreMesh(core_axis_name="c", subcore_axis_name="s")


def _sc_pack_q(z32, sg, dg, e1f, e2f, e3f):
    @functools.partial(
        pl.kernel,
        mesh=_vmesh(),
        compiler_params=pltpu.CompilerParams(needs_layout_passes=False),
        out_type=(jax.ShapeDtypeStruct((_PQ_WORDS,), jnp.int32),
                  jax.ShapeDtypeStruct((64 * 128,), jnp.float32)),
        scratch_types=[
            pltpu.VMEM((_N_NODES,), jnp.int32),
            pltpu.VMEM((_BLK,), jnp.int32),
            pltpu.VMEM((_BLK,), jnp.int32),
            pltpu.VMEM((_LANES,), jnp.int32),
            pltpu.VMEM((128,), jnp.float32),
            pltpu.VMEM((128,), jnp.float32),
            pltpu.VMEM((128,), jnp.float32),
            pltpu.VMEM((512,), jnp.float32),
            pltpu.VMEM((64 * 128,), jnp.float32),
        ],
    )
    def k(z_hbm, sg_hbm, dg_hbm, e1_hbm, e2_hbm, e3_hbm, pq_hbm, tp_hbm,
          zv, sbuf, dbuf, obuf, e1v, e2v, e3v, tv, tpv):
        wid = lax.axis_index("s") * 2 + lax.axis_index("c")

        @pl.when(wid == 0)
        def _():
            pltpu.sync_copy(e1_hbm, e1v)
            pltpu.sync_copy(e2_hbm, e2v)
            pltpu.sync_copy(e3_hbm, e3v)
            for kk in range(8):
                for g in range(4):
                    s = g * 16
                    tv[pl.ds(kk * 64 + s, 16)] = (
                        e1v[pl.ds((kk >> 2) * 64 + s, 16)]
                        + e2v[pl.ds(((kk >> 1) & 1) * 64 + s, 16)]
                        + e3v[pl.ds((kk & 1) * 64 + s, 16)])
            for kk in range(64):
                for g in range(4):
                    s = g * 16
                    tpv[pl.ds(kk * 128 + s, 16)] = tv[pl.ds((kk >> 3) * 64 + s, 16)]
                    tpv[pl.ds(kk * 128 + 64 + s, 16)] = tv[pl.ds((kk & 7) * 64 + s, 16)]
            pltpu.sync_copy(tpv, tp_hbm)

        pltpu.sync_copy(z_hbm, zv)

        def body(i, carry):
            b = wid + _NW * i

            @pl.when(b < _NBLK_G)
            def _():
                off = pl.multiple_of(b * _BLK, _BLK)
                pltpu.sync_copy(sg_hbm.at[pl.ds(off, _BLK)], sbuf)
                pltpu.sync_copy(dg_hbm.at[pl.ds(off, _BLK)], dbuf)
                acc = jnp.zeros((_LANES,), jnp.int32)
                for t in range(16):
                    si = sbuf[pl.ds(t * _LANES, _LANES)]
                    di = dbuf[pl.ds(t * _LANES, _LANES)]
                    zs = plsc.load_gather(zv, [si])
                    zd = plsc.load_gather(zv, [di])
                    q = (zs << 1) | zd
                    acc = acc | (q << (2 * t))
                obuf[...] = acc
                woff = pl.multiple_of(b * _LANES, _LANES)
                pltpu.sync_copy(obuf, pq_hbm.at[pl.ds(woff, _LANES)])

            return carry

        lax.fori_loop(0, _ITERS_G, body, 0)

    return k(z32, sg, dg, e1f, e2f, e3f)


def _sc_expand(pq, sh, dh, tp):
    @functools.partial(
        pl.kernel,
        mesh=_vmesh(),
        compiler_params=pltpu.CompilerParams(needs_layout_passes=False),
        out_type=jax.ShapeDtypeStruct((_NLG // 2, 128), jnp.float32),
        scratch_types=[
            pltpu.VMEM((_PQ_WORDS,), jnp.int32),
            pltpu.VMEM((2 * _BLK,), jnp.int32),
            pltpu.VMEM((2 * _BLK,), jnp.int32),
            pltpu.VMEM((2 * _PAIRS,), jnp.int32),
            pltpu.VMEM((2 * _PAIRS, 128), jnp.float32),
            pltpu.SemaphoreType.DMA((2,)),
            pltpu.SemaphoreType.DMA((2,)),
            pltpu.SemaphoreType.DMA((2,)),
        ],
    )
    def k(pq_hbm, sh_hbm, dh_hbm, tp_hbm, out_hbm,
          pqv, shb, dhb, cb, ob, isem, gsem, osem):
        wid = lax.axis_index("s") * 2 + lax.axis_index("c")
        pltpu.sync_copy(pq_hbm, pqv)
        il = lax.iota(jnp.int32, 16)

        def start_in(slot, b):
            off = pl.multiple_of(b * _BLK, _BLK)
            pltpu.async_copy(sh_hbm.at[pl.ds(off, _BLK)],
                             shb.at[pl.ds(slot * _BLK, _BLK)], isem.at[slot])
            pltpu.async_copy(dh_hbm.at[pl.ds(off, _BLK)],
                             dhb.at[pl.ds(slot * _BLK, _BLK)], isem.at[slot])

        def wait_in(slot, b):
            off = pl.multiple_of(b * _BLK, _BLK)
            pltpu.make_async_copy(sh_hbm.at[pl.ds(off, _BLK)],
                                  shb.at[pl.ds(slot * _BLK, _BLK)],
                                  isem.at[slot]).wait()
            pltpu.make_async_copy(dh_hbm.at[pl.ds(off, _BLK)],
                                  dhb.at[pl.ds(slot * _BLK, _BLK)],
                                  isem.at[slot]).wait()

        def gather_refs(slot):
            idx = cb.at[pl.ds(slot * _PAIRS, _PAIRS)]
            dst = ob.at[pl.ds(slot * _PAIRS, _PAIRS), :]
            return tp_hbm.at[idx], dst

        def start_gather(slot):
            src, dst = gather_refs(slot)
            pltpu.async_copy(src, dst, gsem.at[slot])

        def wait_gather(slot):
            src, dst = gather_refs(slot)
            pltpu.make_async_copy(src, dst, gsem.at[slot]).wait()

        def start_out(slot, b):
            off = pl.multiple_of(b * _PAIRS, _PAIRS)
            pltpu.async_copy(ob.at[pl.ds(slot * _PAIRS, _PAIRS), :],
                             out_hbm.at[pl.ds(off, _PAIRS), :], osem.at[slot])

        def wait_out(slot, b):
            off = pl.multiple_of(b * _PAIRS, _PAIRS)
            pltpu.make_async_copy(ob.at[pl.ds(slot * _PAIRS, _PAIRS), :],
                                  out_hbm.at[pl.ds(off, _PAIRS), :],
                                  osem.at[slot]).wait()

        def unpack(idx):
            w = ((idx >> 8) << 4) | (idx & 15)
            p = plsc.load_gather(pqv, [w])
            return (p >> ((idx >> 3) & 30)) & 3

        def code_of(a, c):
            qa = unpack(a)
            qc = unpack(c)
            za = (qa >> 1) & 1
            zb = qa & 1
            zc = qc & 1
            return (((1 - (za ^ zc)) << 2)
                    | ((1 - (za ^ zb)) << 1)
                    | (1 - (zb ^ zc)))

        start_in(0, wid)

        def body(i, carry):
            par = i & 1
            b = wid + _NW * i
            valid = b < _NBLK_H
            bn = b + _NW

            @pl.when(valid)
            def _():
                wait_in(par, b)

            @pl.when(bn < _NBLK_H)
            def _():
                start_in(1 - par, bn)

            @pl.when(valid)
            def _():
                base = par * _BLK
                for t in range(8):
                    ii = base + 2 * (t * _LANES) + 2 * il
                    a_e = plsc.load_gather(shb, [ii])
                    c_e = plsc.load_gather(dhb, [ii])
                    a_o = plsc.load_gather(shb, [ii + 1])
                    c_o = plsc.load_gather(dhb, [ii + 1])
                    ce = code_of(a_e, c_e)
                    co = code_of(a_o, c_o)
                    cb[pl.ds(par * _PAIRS + t * _LANES, _LANES)] = ce * 8 + co

            @pl.when(jnp.logical_and(valid, i >= 1))
            def _():
                wait_gather(1 - par)
                start_out(1 - par, b - _NW)

            @pl.when(jnp.logical_and(valid, i >= 2))
            def _():
                wait_out(par, b - 2 * _NW)

            @pl.when(valid)
            def _():
                start_gather(par)

            return carry

        lax.fori_loop(0, _ITERS_H, body, 0)

        nv = (_NBLK_H - wid + _NW - 1) // _NW
        last = wid + _NW * (nv - 1)
        lpar = (nv - 1) & 1
        wait_gather(lpar)
        start_out(lpar, last)
        wait_out(1 - lpar, last - _NW)
        wait_out(lpar, last)

    return k(pq, sh, dh, tp)


def kernel(z, edge_index_g, edge_index_h, e1, e2, e3):
    z32 = z.astype(jnp.int32)
    sg = edge_index_g[0].astype(jnp.int32)
    dg = edge_index_g[1].astype(jnp.int32)
    sh = edge_index_h[0].astype(jnp.int32)
    dh = edge_index_h[1].astype(jnp.int32)
    pq, tpf = _sc_pack_q(z32, sg, dg, e1.reshape(128), e2.reshape(128),
                         e3.reshape(128))
    out128 = _sc_expand(pq, sh, dh, tpf.reshape(64, 128))
    return out128.reshape(_NLG, 64)


# speedup vs baseline: 2.2545x; 2.2545x over previous
import functools

import jax
import jax.numpy as jnp
from jax import lax
from jax.experimental import pallas as pl
from jax.experimental.pallas import tpu as pltpu
from jax.experimental.pallas import tpu_sc as plsc

_N_NODES = 50_000
_E = 800_000
_NLG = 800_000
_LANES = 16
_NW = 32
_BLK = 256
_PAIRS = _BLK // 2
_OBW = _PAIRS * 128
_NBLK_G = _E // _BLK
_NBLK_H = _NLG // _BLK
_ITERS_G = (_NBLK_G + _NW - 1) // _NW
_ITERS_H = (_NBLK_H + _NW - 1) // _NW
_PQ_WORDS = _E // _LANES


def _vmesh():
    return plsc.VectorSubcoreMesh(core_axis_name="c", subcore_axis_name="s")


def _sc_pack_q(z32, sg, dg, e1f, e2f, e3f):
    @functools.partial(
        pl.kernel,
        mesh=_vmesh(),
        compiler_params=pltpu.CompilerParams(needs_layout_passes=False),
        out_type=(jax.ShapeDtypeStruct((_PQ_WORDS,), jnp.int32),
                  jax.ShapeDtypeStruct((64 * 128,), jnp.float32)),
        scratch_types=[
            pltpu.VMEM((_N_NODES,), jnp.int32),
            pltpu.VMEM((_BLK,), jnp.int32),
            pltpu.VMEM((_BLK,), jnp.int32),
            pltpu.VMEM((_LANES,), jnp.int32),
            pltpu.VMEM((128,), jnp.float32),
            pltpu.VMEM((128,), jnp.float32),
            pltpu.VMEM((128,), jnp.float32),
            pltpu.VMEM((512,), jnp.float32),
            pltpu.VMEM((64 * 128,), jnp.float32),
        ],
    )
    def k(z_hbm, sg_hbm, dg_hbm, e1_hbm, e2_hbm, e3_hbm, pq_hbm, tp_hbm,
          zv, sbuf, dbuf, obuf, e1v, e2v, e3v, tv, tpv):
        wid = lax.axis_index("s") * 2 + lax.axis_index("c")

        @pl.when(wid == 0)
        def _():
            pltpu.sync_copy(e1_hbm, e1v)
            pltpu.sync_copy(e2_hbm, e2v)
            pltpu.sync_copy(e3_hbm, e3v)
            for kk in range(8):
                for g in range(4):
                    s = g * 16
                    tv[pl.ds(kk * 64 + s, 16)] = (
                        e1v[pl.ds((kk >> 2) * 64 + s, 16)]
                        + e2v[pl.ds(((kk >> 1) & 1) * 64 + s, 16)]
                        + e3v[pl.ds((kk & 1) * 64 + s, 16)])
            for kk in range(64):
                for g in range(4):
                    s = g * 16
                    tpv[pl.ds(kk * 128 + s, 16)] = tv[pl.ds((kk >> 3) * 64 + s, 16)]
                    tpv[pl.ds(kk * 128 + 64 + s, 16)] = tv[pl.ds((kk & 7) * 64 + s, 16)]
            pltpu.sync_copy(tpv, tp_hbm)

        pltpu.sync_copy(z_hbm, zv)

        def body(i, carry):
            b = wid + _NW * i

            @pl.when(b < _NBLK_G)
            def _():
                off = pl.multiple_of(b * _BLK, _BLK)
                pltpu.sync_copy(sg_hbm.at[pl.ds(off, _BLK)], sbuf)
                pltpu.sync_copy(dg_hbm.at[pl.ds(off, _BLK)], dbuf)
                acc = jnp.zeros((_LANES,), jnp.int32)
                for t in range(16):
                    si = sbuf[pl.ds(t * _LANES, _LANES)]
                    di = dbuf[pl.ds(t * _LANES, _LANES)]
                    zs = plsc.load_gather(zv, [si])
                    zd = plsc.load_gather(zv, [di])
                    q = (zs << 1) | zd
                    acc = acc | (q << (2 * t))
                obuf[...] = acc
                woff = pl.multiple_of(b * _LANES, _LANES)
                pltpu.sync_copy(obuf, pq_hbm.at[pl.ds(woff, _LANES)])

            return carry

        lax.fori_loop(0, _ITERS_G, body, 0)

    return k(z32, sg, dg, e1f, e2f, e3f)


def _sc_expand(pq, sh, dh, tp):
    @functools.partial(
        pl.kernel,
        mesh=_vmesh(),
        compiler_params=pltpu.CompilerParams(needs_layout_passes=False),
        out_type=jax.ShapeDtypeStruct((_NLG * 64,), jnp.float32),
        scratch_types=[
            pltpu.VMEM((_PQ_WORDS,), jnp.int32),
            pltpu.VMEM((64 * 128,), jnp.float32),
            pltpu.VMEM((2 * _BLK,), jnp.int32),
            pltpu.VMEM((2 * _BLK,), jnp.int32),
            pltpu.VMEM((2 * _PAIRS,), jnp.int32),
            pltpu.VMEM((2 * _OBW,), jnp.float32),
            pltpu.SemaphoreType.DMA((2,)),
            pltpu.SemaphoreType.DMA((2,)),
        ],
    )
    def k(pq_hbm, sh_hbm, dh_hbm, tp_hbm, out_hbm,
          pqv, tpv, shb, dhb, cb, ob, isem, osem):
        wid = lax.axis_index("s") * 2 + lax.axis_index("c")
        pltpu.sync_copy(pq_hbm, pqv)
        pltpu.sync_copy(tp_hbm, tpv)
        il = lax.iota(jnp.int32, 16)

        def start_in(slot, b):
            off = pl.multiple_of(b * _BLK, _BLK)
            pltpu.async_copy(sh_hbm.at[pl.ds(off, _BLK)],
                             shb.at[pl.ds(slot * _BLK, _BLK)], isem.at[slot])
            pltpu.async_copy(dh_hbm.at[pl.ds(off, _BLK)],
                             dhb.at[pl.ds(slot * _BLK, _BLK)], isem.at[slot])

        def wait_in(slot, b):
            off = pl.multiple_of(b * _BLK, _BLK)
            pltpu.make_async_copy(sh_hbm.at[pl.ds(off, _BLK)],
                                  shb.at[pl.ds(slot * _BLK, _BLK)],
                                  isem.at[slot]).wait()
            pltpu.make_async_copy(dh_hbm.at[pl.ds(off, _BLK)],
                                  dhb.at[pl.ds(slot * _BLK, _BLK)],
                                  isem.at[slot]).wait()

        def start_out(slot, b):
            off = pl.multiple_of(b * _OBW, _OBW)
            pltpu.async_copy(ob.at[pl.ds(slot * _OBW, _OBW)],
                             out_hbm.at[pl.ds(off, _OBW)], osem.at[slot])

        def wait_out(slot, b):
            off = pl.multiple_of(b * _OBW, _OBW)
            pltpu.make_async_copy(ob.at[pl.ds(slot * _OBW, _OBW)],
                                  out_hbm.at[pl.ds(off, _OBW)],
                                  osem.at[slot]).wait()

        def unpack(idx):
            w = ((idx >> 8) << 4) | (idx & 15)
            p = plsc.load_gather(pqv, [w])
            return (p >> ((idx >> 3) & 30)) & 3

        def code_of(a, c):
            qa = unpack(a)
            qc = unpack(c)
            za = (qa >> 1) & 1
            zb = qa & 1
            zc = qc & 1
            return (((1 - (za ^ zc)) << 2)
                    | ((1 - (za ^ zb)) << 1)
                    | (1 - (zb ^ zc)))

        start_in(0, wid)

        def body(i, carry):
            par = i & 1
            b = wid + _NW * i
            valid = b < _NBLK_H
            bn = b + _NW

            @pl.when(valid)
            def _():
                wait_in(par, b)

            @pl.when(bn < _NBLK_H)
            def _():
                start_in(1 - par, bn)

            @pl.when(jnp.logical_and(valid, i >= 2))
            def _():
                wait_out(par, b - 2 * _NW)

            @pl.when(valid)
            def _():
                base = par * _BLK
                for t in range(8):
                    ii = base + 2 * (t * _LANES) + 2 * il
                    a_e = plsc.load_gather(shb, [ii])
                    c_e = plsc.load_gather(dhb, [ii])
                    a_o = plsc.load_gather(shb, [ii + 1])
                    c_o = plsc.load_gather(dhb, [ii + 1])
                    ce = code_of(a_e, c_e)
                    co = code_of(a_o, c_o)
                    cb[pl.ds(par * _PAIRS + t * _LANES, _LANES)] = ce * 8 + co

                obase = par * _OBW

                @plsc.parallel_loop(0, _PAIRS, 16)
                def _jb(m0):
                    vp = cb[pl.ds(par * _PAIRS + m0, 16)]
                    for u in range(16):
                        r = vp[u] << 7
                        d = obase + (m0 + u) * 128
                        for g in range(8):
                            ob[pl.ds(d + g * 16, 16)] = tpv[pl.ds(r + g * 16, 16)]

                start_out(par, b)

            return carry

        lax.fori_loop(0, _ITERS_H, body, 0)

        nv = (_NBLK_H - wid + _NW - 1) // _NW
        last = wid + _NW * (nv - 1)
        wait_out((nv - 1) & 1, last)
        wait_out((nv - 2) & 1, last - _NW)

    return k(pq, sh, dh, tp)


def kernel(z, edge_index_g, edge_index_h, e1, e2, e3):
    z32 = z.astype(jnp.int32)
    sg = edge_index_g[0].astype(jnp.int32)
    dg = edge_index_g[1].astype(jnp.int32)
    sh = edge_index_h[0].astype(jnp.int32)
    dh = edge_index_h[1].astype(jnp.int32)
    pq, tpf = _sc_pack_q(z32, sg, dg, e1.reshape(128), e2.reshape(128),
                         e3.reshape(128))
    flat = _sc_expand(pq, sh, dh, tpf)
    return flat.reshape(_NLG, 64)


# speedup vs baseline: 2.4069x; 1.0676x over previous
import functools

import jax
import jax.numpy as jnp
from jax import lax
from jax.experimental import pallas as pl
from jax.experimental.pallas import tpu as pltpu
from jax.experimental.pallas import tpu_sc as plsc

_N_NODES = 50_000
_E = 800_000
_NLG = 800_000
_LANES = 16
_NW = 32
_BLK = 256
_PAIRS = _BLK // 2
_OBW = _PAIRS * 128
_NBLK_G = _E // _BLK
_NBLK_H = _NLG // _BLK
_ITERS_G = (_NBLK_G + _NW - 1) // _NW
_ITERS_H = (_NBLK_H + _NW - 1) // _NW
_PQ_WORDS = _E // _LANES


def _vmesh():
    return plsc.VectorSubcoreMesh(core_axis_name="c", subcore_axis_name="s")


def _sc_pack_q(z32, sg, dg, e1f, e2f, e3f):
    @functools.partial(
        pl.kernel,
        mesh=_vmesh(),
        compiler_params=pltpu.CompilerParams(needs_layout_passes=False),
        out_type=(jax.ShapeDtypeStruct((_PQ_WORDS,), jnp.int32),
                  jax.ShapeDtypeStruct((64 * 128,), jnp.float32)),
        scratch_types=[
            pltpu.VMEM((_N_NODES,), jnp.int32),
            pltpu.VMEM((2 * _BLK,), jnp.int32),
            pltpu.VMEM((2 * _BLK,), jnp.int32),
            pltpu.VMEM((_ITERS_G * _LANES,), jnp.int32),
            pltpu.VMEM((128,), jnp.float32),
            pltpu.VMEM((128,), jnp.float32),
            pltpu.VMEM((128,), jnp.float32),
            pltpu.VMEM((512,), jnp.float32),
            pltpu.VMEM((64 * 128,), jnp.float32),
            pltpu.SemaphoreType.DMA((2,)),
        ],
    )
    def k(z_hbm, sg_hbm, dg_hbm, e1_hbm, e2_hbm, e3_hbm, pq_hbm, tp_hbm,
          zv, sbuf, dbuf, resb, e1v, e2v, e3v, tv, tpv, isem):
        wid = lax.axis_index("s") * 2 + lax.axis_index("c")

        @pl.when(wid == 0)
        def _():
            pltpu.sync_copy(e1_hbm, e1v)
            pltpu.sync_copy(e2_hbm, e2v)
            pltpu.sync_copy(e3_hbm, e3v)
            for kk in range(8):
                for g in range(4):
                    s = g * 16
                    tv[pl.ds(kk * 64 + s, 16)] = (
                        e1v[pl.ds((kk >> 2) * 64 + s, 16)]
                        + e2v[pl.ds(((kk >> 1) & 1) * 64 + s, 16)]
                        + e3v[pl.ds((kk & 1) * 64 + s, 16)])
            for kk in range(64):
                for g in range(4):
                    s = g * 16
                    tpv[pl.ds(kk * 128 + s, 16)] = tv[pl.ds((kk >> 3) * 64 + s, 16)]
                    tpv[pl.ds(kk * 128 + 64 + s, 16)] = tv[pl.ds((kk & 7) * 64 + s, 16)]
            pltpu.sync_copy(tpv, tp_hbm)

        pltpu.sync_copy(z_hbm, zv)

        nblk = jnp.where(wid < _NBLK_G - 97 * _NW, 98, 97)
        start = wid * 97 + jnp.minimum(wid, _NBLK_G - 97 * _NW)

        def start_in(slot, b):
            off = pl.multiple_of(b * _BLK, _BLK)
            pltpu.async_copy(sg_hbm.at[pl.ds(off, _BLK)],
                             sbuf.at[pl.ds(slot * _BLK, _BLK)], isem.at[slot])
            pltpu.async_copy(dg_hbm.at[pl.ds(off, _BLK)],
                             dbuf.at[pl.ds(slot * _BLK, _BLK)], isem.at[slot])

        def wait_in(slot, b):
            off = pl.multiple_of(b * _BLK, _BLK)
            pltpu.make_async_copy(sg_hbm.at[pl.ds(off, _BLK)],
                                  sbuf.at[pl.ds(slot * _BLK, _BLK)],
                                  isem.at[slot]).wait()
            pltpu.make_async_copy(dg_hbm.at[pl.ds(off, _BLK)],
                                  dbuf.at[pl.ds(slot * _BLK, _BLK)],
                                  isem.at[slot]).wait()

        start_in(0, start)

        def body(i, carry):
            par = i & 1
            b = start + i
            wait_in(par, b)

            @pl.when(i + 1 < nblk)
            def _():
                start_in(1 - par, b + 1)

            base = par * _BLK
            acc = jnp.zeros((_LANES,), jnp.int32)
            for t in range(16):
                si = sbuf[pl.ds(base + t * _LANES, _LANES)]
                di = dbuf[pl.ds(base + t * _LANES, _LANES)]
                zs = plsc.load_gather(zv, [si])
                zd = plsc.load_gather(zv, [di])
                q = (zs << 1) | zd
                acc = acc | (q << (2 * t))
            resb[pl.ds(i * _LANES, _LANES)] = acc
            return carry

        lax.fori_loop(0, nblk, body, 0)

        woff = pl.multiple_of(start * _LANES, _LANES)

        @pl.when(nblk == 98)
        def _():
            pltpu.sync_copy(resb.at[pl.ds(0, 98 * _LANES)],
                            pq_hbm.at[pl.ds(woff, 98 * _LANES)])

        @pl.when(nblk == 97)
        def _():
            pltpu.sync_copy(resb.at[pl.ds(0, 97 * _LANES)],
                            pq_hbm.at[pl.ds(woff, 97 * _LANES)])

    return k(z32, sg, dg, e1f, e2f, e3f)


def _sc_expand(pq, sh, dh, tp):
    @functools.partial(
        pl.kernel,
        mesh=_vmesh(),
        compiler_params=pltpu.CompilerParams(needs_layout_passes=False),
        out_type=jax.ShapeDtypeStruct((_NLG * 64,), jnp.float32),
        scratch_types=[
            pltpu.VMEM((_PQ_WORDS,), jnp.int32),
            pltpu.VMEM((64 * 128,), jnp.float32),
            pltpu.VMEM((2 * _BLK,), jnp.int32),
            pltpu.VMEM((2 * _BLK,), jnp.int32),
            pltpu.VMEM((2 * _PAIRS,), jnp.int32),
            pltpu.VMEM((2 * _OBW,), jnp.float32),
            pltpu.SemaphoreType.DMA((2,)),
            pltpu.SemaphoreType.DMA((2,)),
        ],
    )
    def k(pq_hbm, sh_hbm, dh_hbm, tp_hbm, out_hbm,
          pqv, tpv, shb, dhb, cb, ob, isem, osem):
        wid = lax.axis_index("s") * 2 + lax.axis_index("c")
        pltpu.sync_copy(pq_hbm, pqv)
        pltpu.sync_copy(tp_hbm, tpv)
        il = lax.iota(jnp.int32, 16)

        def start_in(slot, b):
            off = pl.multiple_of(b * _BLK, _BLK)
            pltpu.async_copy(sh_hbm.at[pl.ds(off, _BLK)],
                             shb.at[pl.ds(slot * _BLK, _BLK)], isem.at[slot])
            pltpu.async_copy(dh_hbm.at[pl.ds(off, _BLK)],
                             dhb.at[pl.ds(slot * _BLK, _BLK)], isem.at[slot])

        def wait_in(slot, b):
            off = pl.multiple_of(b * _BLK, _BLK)
            pltpu.make_async_copy(sh_hbm.at[pl.ds(off, _BLK)],
                                  shb.at[pl.ds(slot * _BLK, _BLK)],
                                  isem.at[slot]).wait()
            pltpu.make_async_copy(dh_hbm.at[pl.ds(off, _BLK)],
                                  dhb.at[pl.ds(slot * _BLK, _BLK)],
                                  isem.at[slot]).wait()

        def start_out(slot, b):
            off = pl.multiple_of(b * _OBW, _OBW)
            pltpu.async_copy(ob.at[pl.ds(slot * _OBW, _OBW)],
                             out_hbm.at[pl.ds(off, _OBW)], osem.at[slot])

        def wait_out(slot, b):
            off = pl.multiple_of(b * _OBW, _OBW)
            pltpu.make_async_copy(ob.at[pl.ds(slot * _OBW, _OBW)],
                                  out_hbm.at[pl.ds(off, _OBW)],
                                  osem.at[slot]).wait()

        def unpack(idx):
            w = ((idx >> 8) << 4) | (idx & 15)
            p = plsc.load_gather(pqv, [w])
            return (p >> ((idx >> 3) & 30)) & 3

        def code_of(a, c):
            qa = unpack(a)
            qc = unpack(c)
            za = (qa >> 1) & 1
            zb = qa & 1
            zc = qc & 1
            return (((1 - (za ^ zc)) << 2)
                    | ((1 - (za ^ zb)) << 1)
                    | (1 - (zb ^ zc)))

        start_in(0, wid)

        def body(i, carry):
            par = i & 1
            b = wid + _NW * i
            valid = b < _NBLK_H
            bn = b + _NW

            @pl.when(valid)
            def _():
                wait_in(par, b)

            @pl.when(bn < _NBLK_H)
            def _():
                start_in(1 - par, bn)

            @pl.when(jnp.logical_and(valid, i >= 2))
            def _():
                wait_out(par, b - 2 * _NW)

            @pl.when(valid)
            def _():
                base = par * _BLK
                for t in range(8):
                    ii = base + 2 * (t * _LANES) + 2 * il
                    a_e = plsc.load_gather(shb, [ii])
                    c_e = plsc.load_gather(dhb, [ii])
                    a_o = plsc.load_gather(shb, [ii + 1])
                    c_o = plsc.load_gather(dhb, [ii + 1])
                    ce = code_of(a_e, c_e)
                    co = code_of(a_o, c_o)
                    cb[pl.ds(par * _PAIRS + t * _LANES, _LANES)] = ce * 8 + co

                obase = par * _OBW

                @plsc.parallel_loop(0, _PAIRS, 16)
                def _jb(m0):
                    vp = cb[pl.ds(par * _PAIRS + m0, 16)]
                    for u in range(16):
                        r = vp[u] << 7
                        d = obase + (m0 + u) * 128
                        for g in range(8):
                            ob[pl.ds(d + g * 16, 16)] = tpv[pl.ds(r + g * 16, 16)]

                start_out(par, b)

            return carry

        lax.fori_loop(0, _ITERS_H, body, 0)

        nv = (_NBLK_H - wid + _NW - 1) // _NW
        last = wid + _NW * (nv - 1)
        wait_out((nv - 1) & 1, last)
        wait_out((nv - 2) & 1, last - _NW)

    return k(pq, sh, dh, tp)


def kernel(z, edge_index_g, edge_index_h, e1, e2, e3):
    z32 = z.astype(jnp.int32)
    sg = edge_index_g[0].astype(jnp.int32)
    dg = edge_index_g[1].astype(jnp.int32)
    sh = edge_index_h[0].astype(jnp.int32)
    dh = edge_index_h[1].astype(jnp.int32)
    pq, tpf = _sc_pack_q(z32, sg, dg, e1.reshape(128), e2.reshape(128),
                         e3.reshape(128))
    flat = _sc_expand(pq, sh, dh, tpf)
    return flat.reshape(_NLG, 64)


# speedup vs baseline: 2.7027x; 1.1229x over previous
import functools

import jax
import jax.numpy as jnp
from jax import lax
from jax.experimental import pallas as pl
from jax.experimental.pallas import tpu as pltpu
from jax.experimental.pallas import tpu_sc as plsc

_N_NODES = 50_000
_E = 800_000
_NLG = 800_000
_LANES = 16
_NW = 32
_BLK = 256
_PAIRS = _BLK // 2
_OBW = _PAIRS * 128
_NBLK_G = _E // _BLK
_NBLK_H = _NLG // _BLK
_ITERS_G = (_NBLK_G + _NW - 1) // _NW
_ITERS_H = (_NBLK_H + _NW - 1) // _NW
_PQ_WORDS = _E // _LANES


def _vmesh():
    return plsc.VectorSubcoreMesh(core_axis_name="c", subcore_axis_name="s")


def _sc_pack_q(z32, sg, dg, e1f, e2f, e3f):
    @functools.partial(
        pl.kernel,
        mesh=_vmesh(),
        compiler_params=pltpu.CompilerParams(needs_layout_passes=False),
        out_type=(jax.ShapeDtypeStruct((_PQ_WORDS,), jnp.int32),
                  jax.ShapeDtypeStruct((64 * 128,), jnp.float32)),
        scratch_types=[
            pltpu.VMEM((_N_NODES,), jnp.int32),
            pltpu.VMEM((2 * _BLK,), jnp.int32),
            pltpu.VMEM((2 * _BLK,), jnp.int32),
            pltpu.VMEM((_ITERS_G * _LANES,), jnp.int32),
            pltpu.VMEM((128,), jnp.float32),
            pltpu.VMEM((128,), jnp.float32),
            pltpu.VMEM((128,), jnp.float32),
            pltpu.VMEM((512,), jnp.float32),
            pltpu.VMEM((64 * 128,), jnp.float32),
            pltpu.SemaphoreType.DMA((2,)),
        ],
    )
    def k(z_hbm, sg_hbm, dg_hbm, e1_hbm, e2_hbm, e3_hbm, pq_hbm, tp_hbm,
          zv, sbuf, dbuf, resb, e1v, e2v, e3v, tv, tpv, isem):
        wid = lax.axis_index("s") * 2 + lax.axis_index("c")

        @pl.when(wid == 0)
        def _():
            pltpu.sync_copy(e1_hbm, e1v)
            pltpu.sync_copy(e2_hbm, e2v)
            pltpu.sync_copy(e3_hbm, e3v)
            for kk in range(8):
                for g in range(4):
                    s = g * 16
                    tv[pl.ds(kk * 64 + s, 16)] = (
                        e1v[pl.ds((kk >> 2) * 64 + s, 16)]
                        + e2v[pl.ds(((kk >> 1) & 1) * 64 + s, 16)]
                        + e3v[pl.ds((kk & 1) * 64 + s, 16)])
            for kk in range(64):
                for g in range(4):
                    s = g * 16
                    tpv[pl.ds(kk * 128 + s, 16)] = tv[pl.ds((kk >> 3) * 64 + s, 16)]
                    tpv[pl.ds(kk * 128 + 64 + s, 16)] = tv[pl.ds((kk & 7) * 64 + s, 16)]
            pltpu.sync_copy(tpv, tp_hbm)

        pltpu.sync_copy(z_hbm, zv)

        nblk = jnp.where(wid < _NBLK_G - 97 * _NW, 98, 97)
        start = wid * 97 + jnp.minimum(wid, _NBLK_G - 97 * _NW)

        def start_in(slot, b):
            off = pl.multiple_of(b * _BLK, _BLK)
            pltpu.async_copy(sg_hbm.at[pl.ds(off, _BLK)],
                             sbuf.at[pl.ds(slot * _BLK, _BLK)], isem.at[slot])
            pltpu.async_copy(dg_hbm.at[pl.ds(off, _BLK)],
                             dbuf.at[pl.ds(slot * _BLK, _BLK)], isem.at[slot])

        def wait_in(slot, b):
            off = pl.multiple_of(b * _BLK, _BLK)
            pltpu.make_async_copy(sg_hbm.at[pl.ds(off, _BLK)],
                                  sbuf.at[pl.ds(slot * _BLK, _BLK)],
                                  isem.at[slot]).wait()
            pltpu.make_async_copy(dg_hbm.at[pl.ds(off, _BLK)],
                                  dbuf.at[pl.ds(slot * _BLK, _BLK)],
                                  isem.at[slot]).wait()

        start_in(0, start)

        def body(i, carry):
            par = i & 1
            b = start + i
            wait_in(par, b)

            @pl.when(i + 1 < nblk)
            def _():
                start_in(1 - par, b + 1)

            base = par * _BLK
            acc = jnp.zeros((_LANES,), jnp.int32)
            for t in range(16):
                si = sbuf[pl.ds(base + t * _LANES, _LANES)]
                di = dbuf[pl.ds(base + t * _LANES, _LANES)]
                zs = plsc.load_gather(zv, [si])
                zd = plsc.load_gather(zv, [di])
                q = (zs << 1) | zd
                acc = acc | (q << (2 * t))
            resb[pl.ds(i * _LANES, _LANES)] = acc
            return carry

        lax.fori_loop(0, nblk, body, 0)

        woff = pl.multiple_of(start * _LANES, _LANES)

        @pl.when(nblk == 98)
        def _():
            pltpu.sync_copy(resb.at[pl.ds(0, 98 * _LANES)],
                            pq_hbm.at[pl.ds(woff, 98 * _LANES)])

        @pl.when(nblk == 97)
        def _():
            pltpu.sync_copy(resb.at[pl.ds(0, 97 * _LANES)],
                            pq_hbm.at[pl.ds(woff, 97 * _LANES)])

    return k(z32, sg, dg, e1f, e2f, e3f)


def _sc_expand(pq, sh, dh, tp):
    @functools.partial(
        pl.kernel,
        mesh=_vmesh(),
        compiler_params=pltpu.CompilerParams(needs_layout_passes=False),
        out_type=jax.ShapeDtypeStruct((_NLG * 64,), jnp.float32),
        scratch_types=[
            pltpu.VMEM((_PQ_WORDS,), jnp.int32),
            pltpu.VMEM((64 * 128,), jnp.float32),
            pltpu.VMEM((2 * _BLK,), jnp.int32),
            pltpu.VMEM((2 * _BLK,), jnp.int32),
            pltpu.VMEM((2 * _PAIRS,), jnp.int32),
            pltpu.VMEM((2 * _OBW,), jnp.float32),
            pltpu.SemaphoreType.DMA((2,)),
            pltpu.SemaphoreType.DMA((2,)),
        ],
    )
    def k(pq_hbm, sh_hbm, dh_hbm, tp_hbm, out_hbm,
          pqv, tpv, shb, dhb, cb, ob, isem, osem):
        wid = lax.axis_index("s") * 2 + lax.axis_index("c")
        pltpu.sync_copy(pq_hbm, pqv)
        pltpu.sync_copy(tp_hbm, tpv)
        il = lax.iota(jnp.int32, 16)

        def start_in(slot, b):
            off = pl.multiple_of(b * _BLK, _BLK)
            pltpu.async_copy(sh_hbm.at[pl.ds(off, _BLK)],
                             shb.at[pl.ds(slot * _BLK, _BLK)], isem.at[slot])
            pltpu.async_copy(dh_hbm.at[pl.ds(off, _BLK)],
                             dhb.at[pl.ds(slot * _BLK, _BLK)], isem.at[slot])

        def wait_in(slot, b):
            off = pl.multiple_of(b * _BLK, _BLK)
            pltpu.make_async_copy(sh_hbm.at[pl.ds(off, _BLK)],
                                  shb.at[pl.ds(slot * _BLK, _BLK)],
                                  isem.at[slot]).wait()
            pltpu.make_async_copy(dh_hbm.at[pl.ds(off, _BLK)],
                                  dhb.at[pl.ds(slot * _BLK, _BLK)],
                                  isem.at[slot]).wait()

        def start_out(slot, b):
            off = pl.multiple_of(b * _OBW, _OBW)
            pltpu.async_copy(ob.at[pl.ds(slot * _OBW, _OBW)],
                             out_hbm.at[pl.ds(off, _OBW)], osem.at[slot])

        def wait_out(slot, b):
            off = pl.multiple_of(b * _OBW, _OBW)
            pltpu.make_async_copy(ob.at[pl.ds(slot * _OBW, _OBW)],
                                  out_hbm.at[pl.ds(off, _OBW)],
                                  osem.at[slot]).wait()

        def unpack(idx):
            w = ((idx >> 8) << 4) | (idx & 15)
            p = plsc.load_gather(pqv, [w])
            return (p >> ((idx >> 3) & 30)) & 3

        def code_of(a, c):
            qa = unpack(a)
            qc = unpack(c)
            za = (qa >> 1) & 1
            zb = qa & 1
            zc = qc & 1
            return (((1 - (za ^ zc)) << 2)
                    | ((1 - (za ^ zb)) << 1)
                    | (1 - (zb ^ zc)))

        start_in(0, wid)

        def body(i, carry):
            par = i & 1
            b = wid + _NW * i
            valid = b < _NBLK_H
            bn = b + _NW

            @pl.when(valid)
            def _():
                wait_in(par, b)

            @pl.when(bn < _NBLK_H)
            def _():
                start_in(1 - par, bn)

            @pl.when(jnp.logical_and(valid, i >= 2))
            def _():
                wait_out(par, b - 2 * _NW)

            @pl.when(valid)
            def _():
                base = par * _BLK
                for t in range(8):
                    ii = base + 2 * (t * _LANES) + 2 * il
                    a_e = plsc.load_gather(shb, [ii])
                    c_e = plsc.load_gather(dhb, [ii])
                    a_o = plsc.load_gather(shb, [ii + 1])
                    c_o = plsc.load_gather(dhb, [ii + 1])
                    ce = code_of(a_e, c_e)
                    co = code_of(a_o, c_o)
                    cb[pl.ds(par * _PAIRS + t * _LANES, _LANES)] = ce * 8 + co

                obase = par * _OBW

                @plsc.parallel_loop(0, _PAIRS, 16, unroll=2)
                def _jb(m0):
                    vp = cb[pl.ds(par * _PAIRS + m0, 16)]
                    for u in range(16):
                        r = vp[u] << 7
                        d = obase + (m0 + u) * 128
                        for g in range(8):
                            ob[pl.ds(d + g * 16, 16)] = tpv[pl.ds(r + g * 16, 16)]

                start_out(par, b)

            return carry

        lax.fori_loop(0, _ITERS_H, body, 0)

        nv = (_NBLK_H - wid + _NW - 1) // _NW
        last = wid + _NW * (nv - 1)
        wait_out((nv - 1) & 1, last)
        wait_out((nv - 2) & 1, last - _NW)

    return k(pq, sh, dh, tp)


def kernel(z, edge_index_g, edge_index_h, e1, e2, e3):
    z32 = z.astype(jnp.int32)
    sg = edge_index_g[0].astype(jnp.int32)
    dg = edge_index_g[1].astype(jnp.int32)
    sh = edge_index_h[0].astype(jnp.int32)
    dh = edge_index_h[1].astype(jnp.int32)
    pq, tpf = _sc_pack_q(z32, sg, dg, e1.reshape(128), e2.reshape(128),
                         e3.reshape(128))
    flat = _sc_expand(pq, sh, dh, tpf)
    return flat.reshape(_NLG, 64)
